# baseline (speedup 1.0000x reference)
import functools

import jax
import jax.numpy as jnp
from jax import lax
from jax.experimental import pallas as pl
from jax.experimental.pallas import tpu as pltpu
from jax.experimental.pallas import tpu_sc as plsc

N_NODES = 10000
N_EDGES = 320000
D_FEAT = 128
HEADS1 = 8
OUT1 = 16
N_CLASSES = 7

NC = 2
NS = 16
CHUNK = 104
NCHUNK = 100
PER_TILE = NCHUNK * CHUNK
PER_CORE = NS * PER_TILE
EDGES_PAD = NC * PER_CORE
IDX_ROWS = EDGES_PAD // CHUNK + 2
ZROWS = 632
NACC = NS * ZROWS
W1COLS = 144
W2COLS = 16


def _tc_a_body(x_ref, w1_ref, as_ref, ad_ref, out_ref):
    h = jnp.dot(x_ref[...], w1_ref[...], preferred_element_type=jnp.float32)
    asv = jnp.dot(h, as_ref[...], preferred_element_type=jnp.float32)
    adv = jnp.dot(h, ad_ref[...], preferred_element_type=jnp.float32)
    out_ref[...] = jnp.concatenate([h, asv, adv], axis=1)


def _tc_a(x, w1, as1p, ad1p):
    bn = 2000
    grid = N_NODES // bn
    return pl.pallas_call(
        _tc_a_body,
        grid=(grid,),
        in_specs=[
            pl.BlockSpec((bn, D_FEAT), lambda i: (i, 0)),
            pl.BlockSpec((D_FEAT, D_FEAT), lambda i: (0, 0)),
            pl.BlockSpec((D_FEAT, 16), lambda i: (0, 0)),
            pl.BlockSpec((D_FEAT, 16), lambda i: (0, 0)),
        ],
        out_specs=pl.BlockSpec((bn, 160), lambda i: (i, 0)),
        out_shape=jax.ShapeDtypeStruct((N_NODES, 160), jnp.float32),
    )(x, w1, as1p, ad1p)


def _tc_b_body(acc_ref, expand_ref, b1_ref, w2_ref, a2s_ref, a2d_ref, out_ref):
    s = acc_ref[0] + acc_ref[1]
    num = s[:, :D_FEAT]
    den = s[:, D_FEAT:D_FEAT + HEADS1]
    deninv = 1.0 / (den + 1e-16)
    den128 = jnp.dot(deninv, expand_ref[...],
                     preferred_element_type=jnp.float32)
    o1 = num * den128 + b1_ref[...]
    hmid = jnp.where(o1 > 0, o1, jnp.exp(o1) - 1.0)
    h2 = jnp.dot(hmid, w2_ref[...], preferred_element_type=jnp.float32)
    as2 = jnp.sum(h2 * a2s_ref[...], axis=1, keepdims=True)
    ad2 = jnp.sum(h2 * a2d_ref[...], axis=1, keepdims=True)
    col = lax.broadcasted_iota(jnp.int32, h2.shape, 1)
    t2row = (h2 + (col == 7).astype(jnp.float32)
             + as2 * (col == 8) + ad2 * (col == 9))
    row = (lax.broadcasted_iota(jnp.int32, h2.shape, 0)
           + pl.program_id(0) * h2.shape[0])
    out_ref[...] = jnp.where(row < N_NODES, t2row, 0.0)


def _tc_b(acc1, expand, b1, w2p, a2s, a2d):
    bn = 2528
    grid = NACC // bn
    return pl.pallas_call(
        _tc_b_body,
        grid=(grid,),
        in_specs=[
            pl.BlockSpec((2, bn, W1COLS), lambda i: (0, i, 0)),
            pl.BlockSpec((HEADS1, D_FEAT), lambda i: (0, 0)),
            pl.BlockSpec((1, D_FEAT), lambda i: (0, 0)),
            pl.BlockSpec((D_FEAT, 16), lambda i: (0, 0)),
            pl.BlockSpec((1, 16), lambda i: (0, 0)),
            pl.BlockSpec((1, 16), lambda i: (0, 0)),
        ],
        out_specs=pl.BlockSpec((bn, 16), lambda i: (i, 0)),
        out_shape=jax.ShapeDtypeStruct((NACC, 16), jnp.float32),
    )(acc1, expand, b1, w2p, a2s, a2d)


def _tc_c_body(acc_ref, b2_ref, out_ref):
    s = acc_ref[0] + acc_ref[1]
    den = s[:, 7:8]
    o = s * (1.0 / (den + 1e-16)) + b2_ref[...]
    col = lax.broadcasted_iota(jnp.int32, o.shape, 1)
    valid = col < N_CLASSES
    om = jnp.where(valid, o, -jnp.inf)
    m = jnp.max(om, axis=1, keepdims=True)
    ex = jnp.where(valid, jnp.exp(o - m), 0.0)
    lse = jnp.log(jnp.sum(ex, axis=1, keepdims=True))
    out_ref[...] = (o - m) - lse


def _tc_c(acc2, b2p):
    bn = 2528
    grid = NACC // bn
    return pl.pallas_call(
        _tc_c_body,
        grid=(grid,),
        in_specs=[
            pl.BlockSpec((2, bn, W2COLS), lambda i: (0, i, 0)),
            pl.BlockSpec((1, 16), lambda i: (0, 0)),
        ],
        out_specs=pl.BlockSpec((bn, 16), lambda i: (i, 0)),
        out_shape=jax.ShapeDtypeStruct((NACC, 16), jnp.float32),
    )(acc2, b2p)


def _make_edge_pass(ws, wd, compute_edge, unroll, nch0=NCHUNK, nch1=NCHUNK):
    def body(tsrc_hbm, tdst_hbm, src2_hbm, dst2_hbm, zeros_hbm, out_hbm,
             idxSA, idxDA, idxSB, idxDB, tsA, tdA, tsB, tdB,
             acc, semA, semB, semI, semSA, semSB):
        c = lax.axis_index("c")
        s = lax.axis_index("s")
        row0 = jnp.where(c == 0, s * nch0, NS * nch0 + s * nch1)
        nq = jnp.where(c == 0, nch0 // 4, nch1 // 4)

        pltpu.sync_copy(zeros_hbm, acc.at[pl.ds(s * ZROWS, ZROWS)])
        pltpu.sync_copy(src2_hbm.at[pl.ds(row0, 2)], idxSA)
        pltpu.sync_copy(dst2_hbm.at[pl.ds(row0, 2)], idxDA)
        plsc.subcore_barrier()

        def gather(sidx_row, didx_row, ts, td, sem):
            h1 = pltpu.async_copy(tsrc_hbm.at[sidx_row], ts, sem)
            h2 = pltpu.async_copy(tdst_hbm.at[didx_row], td, sem)
            return h1, h2

        def drain(ts, td, sem):
            pltpu.make_async_copy(tsrc_hbm.at[pl.ds(0, CHUNK)], ts, sem).wait()
            pltpu.make_async_copy(tdst_hbm.at[pl.ds(0, CHUNK)], td, sem).wait()

        def drain_scatter(ts, sem):
            pltpu.make_async_copy(tsrc_hbm.at[pl.ds(0, CHUNK)], ts, sem).wait()

        def compute(ts, td, didx_row, semS):
            @plsc.parallel_loop(0, CHUNK, 1, unroll=unroll)
            def edge_body(e):
                compute_edge(ts, td, e)
            return pltpu.async_copy(ts, acc.at[didx_row], semS, add=True)

        gather(idxSA.at[0], idxDA.at[0], tsA, tdA, semA)
        pltpu.async_copy(tsrc_hbm.at[pl.ds(0, CHUNK)], tsB, semSB)

        def quad_body(j, _):
            q0 = 4 * j
            fb1 = pltpu.async_copy(src2_hbm.at[pl.ds(row0 + q0 + 2, 2)],
                                   idxSB, semI)
            fb2 = pltpu.async_copy(dst2_hbm.at[pl.ds(row0 + q0 + 2, 2)],
                                   idxDB, semI)
            drain(tsA, tdA, semA)
            drain_scatter(tsB, semSB)
            g1a, g1b = gather(idxSA.at[1], idxDA.at[1], tsB, tdB, semB)
            sA0 = compute(tsA, tdA, idxDA.at[0], semSA)
            g1a.wait(); g1b.wait()
            fb1.wait(); fb2.wait()
            sA0.wait()
            g2a, g2b = gather(idxSB.at[0], idxDB.at[0], tsA, tdA, semA)
            sB1 = compute(tsB, tdB, idxDA.at[1], semSB)
            fa1 = pltpu.async_copy(src2_hbm.at[pl.ds(row0 + q0 + 4, 2)],
                                   idxSA, semI)
            fa2 = pltpu.async_copy(dst2_hbm.at[pl.ds(row0 + q0 + 4, 2)],
                                   idxDA, semI)
            g2a.wait(); g2b.wait()
            sB1.wait()
            g3a, g3b = gather(idxSB.at[1], idxDB.at[1], tsB, tdB, semB)
            sA2 = compute(tsA, tdA, idxDB.at[0], semSA)
            fa1.wait(); fa2.wait()
            g3a.wait(); g3b.wait()
            sA2.wait()
            gather(idxSA.at[0], idxDA.at[0], tsA, tdA, semA)
            compute(tsB, tdB, idxDB.at[1], semSB)
            return ()

        lax.fori_loop(0, nq, quad_body, ())
        drain(tsA, tdA, semA)
        drain_scatter(tsB, semSB)
        plsc.subcore_barrier()
        pltpu.sync_copy(acc.at[pl.ds(s * ZROWS, ZROWS)],
                        out_hbm.at[c, pl.ds(s * ZROWS, ZROWS)])

    def call(tsrc, tdst, src2, dst2, zeros):
        mesh = plsc.VectorSubcoreMesh(core_axis_name="c", subcore_axis_name="s")
        return pl.kernel(
            body,
            out_type=jax.ShapeDtypeStruct((NC, NACC, ws), jnp.float32),
            mesh=mesh,
            compiler_params=pltpu.CompilerParams(use_tc_tiling_on_sc=False),
            scratch_types=[
                pltpu.VMEM((2, CHUNK), jnp.int32),
                pltpu.VMEM((2, CHUNK), jnp.int32),
                pltpu.VMEM((2, CHUNK), jnp.int32),
                pltpu.VMEM((2, CHUNK), jnp.int32),
                pltpu.VMEM((CHUNK, ws), jnp.float32),
                pltpu.VMEM((CHUNK, wd), jnp.float32),
                pltpu.VMEM((CHUNK, ws), jnp.float32),
                pltpu.VMEM((CHUNK, wd), jnp.float32),
                pltpu.VMEM_SHARED((NACC, ws), jnp.float32),
                pltpu.SemaphoreType.DMA,
                pltpu.SemaphoreType.DMA,
                pltpu.SemaphoreType.DMA,
                pltpu.SemaphoreType.DMA,
                pltpu.SemaphoreType.DMA,
            ],
        )(tsrc, tdst, src2, dst2, zeros)

    return call


_KIDX = None


def _edge1(ts, td, e):
    av = ts[e, pl.ds(D_FEAT, 16)] + td[e, :]
    ev = jnp.where(av > 0, av, 0.2 * av)
    w = jnp.exp(ev)
    for k in range(HEADS1):
        kidx = jnp.full((16,), k, jnp.int32)
        wk = w.at[kidx].get(mode="promise_in_bounds")
        ts[e, pl.ds(k * 16, 16)] = ts[e, pl.ds(k * 16, 16)] * wk
    ts[e, pl.ds(D_FEAT, 16)] = w


def _edge2(ts, td, e):
    sv = ts[e, :]
    dv = td[e, :]
    k8 = jnp.full((16,), 8, jnp.int32)
    k9 = jnp.full((16,), 9, jnp.int32)
    a8 = sv.at[k8].get(mode="promise_in_bounds")
    a9 = dv.at[k9].get(mode="promise_in_bounds")
    av = a8 + a9
    ev = jnp.where(av > 0, av, 0.2 * av)
    w = jnp.exp(ev)
    ts[e, :] = sv * w


_sc1_call = _make_edge_pass(W1COLS, 16, _edge1, 4, nch0=80, nch1=120)
_sc2_call = _make_edge_pass(W2COLS, W2COLS, _edge2, 8, nch0=92, nch1=108)


def _sc1(t1s, t1d, src2, dst2, zeros1):
    return _sc1_call(t1s, t1d, src2, dst2, zeros1)


def _sc2(t2, src2, dst2, zeros2):
    return _sc2_call(t2, t2, src2, dst2, zeros2)


def kernel(x, edge_index, W1, a_src1, a_dst1, b1, W2, a_src2, a_dst2, b2):
    n = N_NODES
    loops = jnp.arange(n, dtype=jnp.int32)
    npad = IDX_ROWS * CHUNK - N_EDGES - n
    dummy = jnp.full((npad,), n, jnp.int32)
    src = jnp.concatenate([edge_index[0].astype(jnp.int32), loops, dummy])
    dst = jnp.concatenate([edge_index[1].astype(jnp.int32), loops, dummy])
    src = src.reshape(IDX_ROWS, CHUNK)
    dst = dst.reshape(IDX_ROWS, CHUNK)

    rows = jnp.arange(D_FEAT)
    heads = rows // OUT1
    sel = (jnp.arange(16)[None, :] == heads[:, None]).astype(jnp.float32)
    as1p = sel * a_src1.reshape(-1)[:, None]
    ad1p = sel * a_dst1.reshape(-1)[:, None]

    nodes = _tc_a(x, W1, as1p, ad1p)
    t1s = jnp.pad(nodes[:, :W1COLS], ((0, 1), (0, 0)))
    t1d = jnp.pad(nodes[:, W1COLS:160], ((0, 1), (0, 0)))

    zeros1 = jnp.zeros((ZROWS, W1COLS), jnp.float32)
    acc1 = _sc1(t1s, t1d, src, dst, zeros1)

    expand = (jnp.arange(D_FEAT)[None, :] // OUT1 ==
              jnp.arange(HEADS1)[:, None]).astype(jnp.float32)
    w2p = jnp.pad(W2, ((0, 0), (0, 16 - N_CLASSES)))
    a2s = jnp.pad(a_src2, ((0, 0), (0, 16 - N_CLASSES)))
    a2d = jnp.pad(a_dst2, ((0, 0), (0, 16 - N_CLASSES)))
    t2 = _tc_b(acc1, expand, b1.reshape(1, -1), w2p, a2s, a2d)

    zeros2 = jnp.zeros((ZROWS, W2COLS), jnp.float32)
    acc2 = _sc2(t2[:n + 1], src, dst, zeros2)

    b2p = jnp.pad(b2, (0, 16 - N_CLASSES)).reshape(1, 16)
    out = _tc_c(acc2, b2p)
    return out[:n, :N_CLASSES]


# speedup vs baseline: 1.1043x; 1.1043x over previous
import functools

import jax
import jax.numpy as jnp
from jax import lax
from jax.experimental import pallas as pl
from jax.experimental.pallas import tpu as pltpu
from jax.experimental.pallas import tpu_sc as plsc

N_NODES = 10000
N_EDGES = 320000
D_FEAT = 128
HEADS1 = 8
OUT1 = 16
N_CLASSES = 7

NC = 2
NS = 16
CHUNK = 104
NCHUNK = 100
PER_TILE = NCHUNK * CHUNK
PER_CORE = NS * PER_TILE
EDGES_PAD = NC * PER_CORE
IDX_ROWS = EDGES_PAD // CHUNK + 2
ZROWS = 632
NACC = NS * ZROWS
W1COLS = 144
W2COLS = 16


def _tc_a_body(x_ref, w1_ref, as_ref, ad_ref, out_ref):
    h = jnp.dot(x_ref[...], w1_ref[...], preferred_element_type=jnp.float32)
    asv = jnp.dot(h, as_ref[...], preferred_element_type=jnp.float32)
    adv = jnp.dot(h, ad_ref[...], preferred_element_type=jnp.float32)
    out_ref[...] = jnp.concatenate([h, asv, adv], axis=1)


def _tc_a(x, w1, as1p, ad1p):
    bn = 2000
    grid = N_NODES // bn
    return pl.pallas_call(
        _tc_a_body,
        grid=(grid,),
        in_specs=[
            pl.BlockSpec((bn, D_FEAT), lambda i: (i, 0)),
            pl.BlockSpec((D_FEAT, D_FEAT), lambda i: (0, 0)),
            pl.BlockSpec((D_FEAT, 16), lambda i: (0, 0)),
            pl.BlockSpec((D_FEAT, 16), lambda i: (0, 0)),
        ],
        out_specs=pl.BlockSpec((bn, 160), lambda i: (i, 0)),
        out_shape=jax.ShapeDtypeStruct((N_NODES, 160), jnp.float32),
    )(x, w1, as1p, ad1p)


def _tc_b_body(acc_ref, expand_ref, b1_ref, w2_ref, a2s_ref, a2d_ref, out_ref):
    s = acc_ref[0] + acc_ref[1]
    num = s[:, :D_FEAT]
    den = s[:, D_FEAT:D_FEAT + HEADS1]
    deninv = 1.0 / (den + 1e-16)
    den128 = jnp.dot(deninv, expand_ref[...],
                     preferred_element_type=jnp.float32)
    o1 = num * den128 + b1_ref[...]
    hmid = jnp.where(o1 > 0, o1, jnp.exp(o1) - 1.0)
    h2 = jnp.dot(hmid, w2_ref[...], preferred_element_type=jnp.float32)
    as2 = jnp.sum(h2 * a2s_ref[...], axis=1, keepdims=True)
    ad2 = jnp.sum(h2 * a2d_ref[...], axis=1, keepdims=True)
    col = lax.broadcasted_iota(jnp.int32, h2.shape, 1)
    t2row = (h2 + (col == 7).astype(jnp.float32)
             + as2 * (col == 8) + ad2 * (col == 9))
    row = (lax.broadcasted_iota(jnp.int32, h2.shape, 0)
           + pl.program_id(0) * h2.shape[0])
    out_ref[...] = jnp.where(row < N_NODES, t2row, 0.0)


def _tc_b(acc1, expand, b1, w2p, a2s, a2d):
    bn = 2528
    grid = NACC // bn
    return pl.pallas_call(
        _tc_b_body,
        grid=(grid,),
        in_specs=[
            pl.BlockSpec((2, bn, W1COLS), lambda i: (0, i, 0)),
            pl.BlockSpec((HEADS1, D_FEAT), lambda i: (0, 0)),
            pl.BlockSpec((1, D_FEAT), lambda i: (0, 0)),
            pl.BlockSpec((D_FEAT, 16), lambda i: (0, 0)),
            pl.BlockSpec((1, 16), lambda i: (0, 0)),
            pl.BlockSpec((1, 16), lambda i: (0, 0)),
        ],
        out_specs=pl.BlockSpec((bn, 16), lambda i: (i, 0)),
        out_shape=jax.ShapeDtypeStruct((NACC, 16), jnp.float32),
    )(acc1, expand, b1, w2p, a2s, a2d)


def _tc_c_body(acc_ref, b2_ref, out_ref):
    s = acc_ref[0] + acc_ref[1]
    den = s[:, 7:8]
    o = s * (1.0 / (den + 1e-16)) + b2_ref[...]
    col = lax.broadcasted_iota(jnp.int32, o.shape, 1)
    valid = col < N_CLASSES
    om = jnp.where(valid, o, -jnp.inf)
    m = jnp.max(om, axis=1, keepdims=True)
    ex = jnp.where(valid, jnp.exp(o - m), 0.0)
    lse = jnp.log(jnp.sum(ex, axis=1, keepdims=True))
    out_ref[...] = (o - m) - lse


def _tc_c(acc2, b2p):
    bn = 2528
    grid = NACC // bn
    return pl.pallas_call(
        _tc_c_body,
        grid=(grid,),
        in_specs=[
            pl.BlockSpec((2, bn, W2COLS), lambda i: (0, i, 0)),
            pl.BlockSpec((1, 16), lambda i: (0, 0)),
        ],
        out_specs=pl.BlockSpec((bn, 16), lambda i: (i, 0)),
        out_shape=jax.ShapeDtypeStruct((NACC, 16), jnp.float32),
    )(acc2, b2p)


def _make_edge_pass(ws, wd, compute_edge, unroll, nch0=NCHUNK, nch1=NCHUNK):
    def body(tsrc_hbm, tdst_hbm, src2_hbm, dst2_hbm, zeros_hbm, out_hbm,
             idxSA, idxDA, idxSB, idxDB, tsA, tdA, tsB, tdB,
             acc, semA, semB, semI, semSA, semSB):
        c = lax.axis_index("c")
        s = lax.axis_index("s")
        row0 = jnp.where(c == 0, s * nch0, NS * nch0 + s * nch1)
        nq = jnp.where(c == 0, nch0 // 4, nch1 // 4)

        pltpu.sync_copy(zeros_hbm, acc.at[pl.ds(s * ZROWS, ZROWS)])
        pltpu.sync_copy(src2_hbm.at[pl.ds(row0, 2)], idxSA)
        pltpu.sync_copy(dst2_hbm.at[pl.ds(row0, 2)], idxDA)
        plsc.subcore_barrier()

        def gather(sidx_row, didx_row, ts, td, sem):
            h1 = pltpu.async_copy(tsrc_hbm.at[sidx_row], ts, sem)
            h2 = pltpu.async_copy(tdst_hbm.at[didx_row], td, sem)
            return h1, h2

        def drain(ts, td, sem):
            pltpu.make_async_copy(tsrc_hbm.at[pl.ds(0, CHUNK)], ts, sem).wait()
            pltpu.make_async_copy(tdst_hbm.at[pl.ds(0, CHUNK)], td, sem).wait()

        def drain_scatter(ts, sem):
            pltpu.make_async_copy(tsrc_hbm.at[pl.ds(0, CHUNK)], ts, sem).wait()

        def compute(ts, td, didx_row, semS):
            @plsc.parallel_loop(0, CHUNK, 1, unroll=unroll)
            def edge_body(e):
                compute_edge(ts, td, e)
            return pltpu.async_copy(ts, acc.at[didx_row], semS, add=True)

        gather(idxSA.at[0], idxDA.at[0], tsA, tdA, semA)
        pltpu.async_copy(tsrc_hbm.at[pl.ds(0, CHUNK)], tsB, semSB)

        def quad_body(j, _):
            q0 = 4 * j
            fb1 = pltpu.async_copy(src2_hbm.at[pl.ds(row0 + q0 + 2, 2)],
                                   idxSB, semI)
            fb2 = pltpu.async_copy(dst2_hbm.at[pl.ds(row0 + q0 + 2, 2)],
                                   idxDB, semI)
            drain(tsA, tdA, semA)
            drain_scatter(tsB, semSB)
            g1a, g1b = gather(idxSA.at[1], idxDA.at[1], tsB, tdB, semB)
            sA0 = compute(tsA, tdA, idxDA.at[0], semSA)
            g1a.wait(); g1b.wait()
            fb1.wait(); fb2.wait()
            sA0.wait()
            g2a, g2b = gather(idxSB.at[0], idxDB.at[0], tsA, tdA, semA)
            sB1 = compute(tsB, tdB, idxDA.at[1], semSB)
            fa1 = pltpu.async_copy(src2_hbm.at[pl.ds(row0 + q0 + 4, 2)],
                                   idxSA, semI)
            fa2 = pltpu.async_copy(dst2_hbm.at[pl.ds(row0 + q0 + 4, 2)],
                                   idxDA, semI)
            g2a.wait(); g2b.wait()
            sB1.wait()
            g3a, g3b = gather(idxSB.at[1], idxDB.at[1], tsB, tdB, semB)
            sA2 = compute(tsA, tdA, idxDB.at[0], semSA)
            fa1.wait(); fa2.wait()
            g3a.wait(); g3b.wait()
            sA2.wait()
            gather(idxSA.at[0], idxDA.at[0], tsA, tdA, semA)
            compute(tsB, tdB, idxDB.at[1], semSB)
            return ()

        lax.fori_loop(0, nq, quad_body, ())
        drain(tsA, tdA, semA)
        drain_scatter(tsB, semSB)
        plsc.subcore_barrier()
        pltpu.sync_copy(acc.at[pl.ds(s * ZROWS, ZROWS)],
                        out_hbm.at[c, pl.ds(s * ZROWS, ZROWS)])

    def call(tsrc, tdst, src2, dst2, zeros):
        mesh = plsc.VectorSubcoreMesh(core_axis_name="c", subcore_axis_name="s")
        return pl.kernel(
            body,
            out_type=jax.ShapeDtypeStruct((NC, NACC, ws), jnp.float32),
            mesh=mesh,
            compiler_params=pltpu.CompilerParams(use_tc_tiling_on_sc=False),
            scratch_types=[
                pltpu.VMEM((2, CHUNK), jnp.int32),
                pltpu.VMEM((2, CHUNK), jnp.int32),
                pltpu.VMEM((2, CHUNK), jnp.int32),
                pltpu.VMEM((2, CHUNK), jnp.int32),
                pltpu.VMEM((CHUNK, ws), jnp.float32),
                pltpu.VMEM((CHUNK, wd), jnp.float32),
                pltpu.VMEM((CHUNK, ws), jnp.float32),
                pltpu.VMEM((CHUNK, wd), jnp.float32),
                pltpu.VMEM_SHARED((NACC, ws), jnp.float32),
                pltpu.SemaphoreType.DMA,
                pltpu.SemaphoreType.DMA,
                pltpu.SemaphoreType.DMA,
                pltpu.SemaphoreType.DMA,
                pltpu.SemaphoreType.DMA,
            ],
        )(tsrc, tdst, src2, dst2, zeros)

    return call


_KIDX = None


def _edge1(ts, td, e):
    av = ts[e, pl.ds(D_FEAT, 16)] + td[e, :]
    ev = jnp.where(av > 0, av, 0.2 * av)
    w = jnp.exp(ev)
    for k in range(HEADS1):
        kidx = jnp.full((16,), k, jnp.int32)
        wk = w.at[kidx].get(mode="promise_in_bounds")
        ts[e, pl.ds(k * 16, 16)] = ts[e, pl.ds(k * 16, 16)] * wk
    ts[e, pl.ds(D_FEAT, 16)] = w


def _edge2(ts, td, e):
    sv = ts[e, :]
    dv = td[e, :]
    k8 = jnp.full((16,), 8, jnp.int32)
    k9 = jnp.full((16,), 9, jnp.int32)
    a8 = sv.at[k8].get(mode="promise_in_bounds")
    a9 = dv.at[k9].get(mode="promise_in_bounds")
    av = a8 + a9
    ev = jnp.where(av > 0, av, 0.2 * av)
    w = jnp.exp(ev)
    ts[e, :] = sv * w


_sc1_call = _make_edge_pass(W1COLS, 16, _edge1, 4, nch0=120, nch1=80)
_sc2_call = _make_edge_pass(W2COLS, W2COLS, _edge2, 8, nch0=108, nch1=92)


def _sc1(t1s, t1d, src2, dst2, zeros1):
    return _sc1_call(t1s, t1d, src2, dst2, zeros1)


def _sc2(t2, src2, dst2, zeros2):
    return _sc2_call(t2, t2, src2, dst2, zeros2)


def kernel(x, edge_index, W1, a_src1, a_dst1, b1, W2, a_src2, a_dst2, b2):
    n = N_NODES
    loops = jnp.arange(n, dtype=jnp.int32)
    npad = IDX_ROWS * CHUNK - N_EDGES - n
    dummy = jnp.full((npad,), n, jnp.int32)
    src = jnp.concatenate([edge_index[0].astype(jnp.int32), loops, dummy])
    dst = jnp.concatenate([edge_index[1].astype(jnp.int32), loops, dummy])
    src = src.reshape(IDX_ROWS, CHUNK)
    dst = dst.reshape(IDX_ROWS, CHUNK)

    rows = jnp.arange(D_FEAT)
    heads = rows // OUT1
    sel = (jnp.arange(16)[None, :] == heads[:, None]).astype(jnp.float32)
    as1p = sel * a_src1.reshape(-1)[:, None]
    ad1p = sel * a_dst1.reshape(-1)[:, None]

    nodes = _tc_a(x, W1, as1p, ad1p)
    t1s = jnp.pad(nodes[:, :W1COLS], ((0, 1), (0, 0)))
    t1d = jnp.pad(nodes[:, W1COLS:160], ((0, 1), (0, 0)))

    zeros1 = jnp.zeros((ZROWS, W1COLS), jnp.float32)
    acc1 = _sc1(t1s, t1d, src, dst, zeros1)

    expand = (jnp.arange(D_FEAT)[None, :] // OUT1 ==
              jnp.arange(HEADS1)[:, None]).astype(jnp.float32)
    w2p = jnp.pad(W2, ((0, 0), (0, 16 - N_CLASSES)))
    a2s = jnp.pad(a_src2, ((0, 0), (0, 16 - N_CLASSES)))
    a2d = jnp.pad(a_dst2, ((0, 0), (0, 16 - N_CLASSES)))
    t2 = _tc_b(acc1, expand, b1.reshape(1, -1), w2p, a2s, a2d)

    zeros2 = jnp.zeros((ZROWS, W2COLS), jnp.float32)
    acc2 = _sc2(t2[:n + 1], src, dst, zeros2)

    b2p = jnp.pad(b2, (0, 16 - N_CLASSES)).reshape(1, 16)
    out = _tc_c(acc2, b2p)
    return out[:n, :N_CLASSES]


# speedup vs baseline: 1.1287x; 1.0221x over previous
import functools

import jax
import jax.numpy as jnp
from jax import lax
from jax.experimental import pallas as pl
from jax.experimental.pallas import tpu as pltpu
from jax.experimental.pallas import tpu_sc as plsc

N_NODES = 10000
N_EDGES = 320000
D_FEAT = 128
HEADS1 = 8
OUT1 = 16
N_CLASSES = 7

NC = 2
NS = 16
CHUNK = 104
NCHUNK = 100
PER_TILE = NCHUNK * CHUNK
PER_CORE = NS * PER_TILE
EDGES_PAD = NC * PER_CORE
IDX_ROWS = EDGES_PAD // CHUNK + 2
ZROWS = 632
NACC = NS * ZROWS
W1COLS = 144
W2COLS = 16


def _tc_a_body(x_ref, w1_ref, as_ref, ad_ref, out_ref):
    h = jnp.dot(x_ref[...], w1_ref[...], preferred_element_type=jnp.float32)
    asv = jnp.dot(h, as_ref[...], preferred_element_type=jnp.float32)
    adv = jnp.dot(h, ad_ref[...], preferred_element_type=jnp.float32)
    out_ref[...] = jnp.concatenate([h, asv, adv], axis=1)


def _tc_a(x, w1, as1p, ad1p):
    bn = 2000
    grid = N_NODES // bn
    return pl.pallas_call(
        _tc_a_body,
        grid=(grid,),
        in_specs=[
            pl.BlockSpec((bn, D_FEAT), lambda i: (i, 0)),
            pl.BlockSpec((D_FEAT, D_FEAT), lambda i: (0, 0)),
            pl.BlockSpec((D_FEAT, 16), lambda i: (0, 0)),
            pl.BlockSpec((D_FEAT, 16), lambda i: (0, 0)),
        ],
        out_specs=pl.BlockSpec((bn, 160), lambda i: (i, 0)),
        out_shape=jax.ShapeDtypeStruct((N_NODES, 160), jnp.float32),
    )(x, w1, as1p, ad1p)


def _tc_b_body(acc_ref, expand_ref, b1_ref, w2_ref, a2s_ref, a2d_ref, out_ref):
    s = acc_ref[0] + acc_ref[1]
    num = s[:, :D_FEAT]
    den = s[:, D_FEAT:D_FEAT + HEADS1]
    deninv = 1.0 / (den + 1e-16)
    den128 = jnp.dot(deninv, expand_ref[...],
                     preferred_element_type=jnp.float32)
    o1 = num * den128 + b1_ref[...]
    hmid = jnp.where(o1 > 0, o1, jnp.exp(o1) - 1.0)
    h2 = jnp.dot(hmid, w2_ref[...], preferred_element_type=jnp.float32)
    as2 = jnp.sum(h2 * a2s_ref[...], axis=1, keepdims=True)
    ad2 = jnp.sum(h2 * a2d_ref[...], axis=1, keepdims=True)
    col = lax.broadcasted_iota(jnp.int32, h2.shape, 1)
    t2row = (h2 + (col == 7).astype(jnp.float32)
             + as2 * (col == 8) + ad2 * (col == 9))
    row = (lax.broadcasted_iota(jnp.int32, h2.shape, 0)
           + pl.program_id(0) * h2.shape[0])
    out_ref[...] = jnp.where(row < N_NODES, t2row, 0.0)


def _tc_b(acc1, expand, b1, w2p, a2s, a2d):
    bn = 2528
    grid = NACC // bn
    return pl.pallas_call(
        _tc_b_body,
        grid=(grid,),
        in_specs=[
            pl.BlockSpec((2, bn, W1COLS), lambda i: (0, i, 0)),
            pl.BlockSpec((HEADS1, D_FEAT), lambda i: (0, 0)),
            pl.BlockSpec((1, D_FEAT), lambda i: (0, 0)),
            pl.BlockSpec((D_FEAT, 16), lambda i: (0, 0)),
            pl.BlockSpec((1, 16), lambda i: (0, 0)),
            pl.BlockSpec((1, 16), lambda i: (0, 0)),
        ],
        out_specs=pl.BlockSpec((bn, 16), lambda i: (i, 0)),
        out_shape=jax.ShapeDtypeStruct((NACC, 16), jnp.float32),
    )(acc1, expand, b1, w2p, a2s, a2d)


def _tc_c_body(acc_ref, b2_ref, out_ref):
    s = acc_ref[0] + acc_ref[1]
    den = s[:, 7:8]
    o = s * (1.0 / (den + 1e-16)) + b2_ref[...]
    col = lax.broadcasted_iota(jnp.int32, o.shape, 1)
    valid = col < N_CLASSES
    om = jnp.where(valid, o, -jnp.inf)
    m = jnp.max(om, axis=1, keepdims=True)
    ex = jnp.where(valid, jnp.exp(o - m), 0.0)
    lse = jnp.log(jnp.sum(ex, axis=1, keepdims=True))
    out_ref[...] = (o - m) - lse


def _tc_c(acc2, b2p):
    bn = 2528
    grid = NACC // bn
    return pl.pallas_call(
        _tc_c_body,
        grid=(grid,),
        in_specs=[
            pl.BlockSpec((2, bn, W2COLS), lambda i: (0, i, 0)),
            pl.BlockSpec((1, 16), lambda i: (0, 0)),
        ],
        out_specs=pl.BlockSpec((bn, 16), lambda i: (i, 0)),
        out_shape=jax.ShapeDtypeStruct((NACC, 16), jnp.float32),
    )(acc2, b2p)


def _make_edge_pass(ws, wd, compute_edge, unroll, nch0=NCHUNK, nch1=NCHUNK):
    def body(tsrc_hbm, tdst_hbm, src2_hbm, dst2_hbm, zeros_hbm, out_hbm,
             idxSA, idxDA, idxSB, idxDB, tsA, tdA, tsB, tdB,
             acc, semA, semB, semI, semSA, semSB):
        c = lax.axis_index("c")
        s = lax.axis_index("s")
        row0 = jnp.where(c == 0, s * nch0, NS * nch0 + s * nch1)
        nq = jnp.where(c == 0, nch0 // 4, nch1 // 4)

        pltpu.sync_copy(zeros_hbm, acc.at[pl.ds(s * ZROWS, ZROWS)])
        pltpu.sync_copy(src2_hbm.at[pl.ds(row0, 2)], idxSA)
        pltpu.sync_copy(dst2_hbm.at[pl.ds(row0, 2)], idxDA)
        plsc.subcore_barrier()

        def gather(sidx_row, didx_row, ts, td, sem):
            h1 = pltpu.async_copy(tsrc_hbm.at[sidx_row], ts, sem)
            h2 = pltpu.async_copy(tdst_hbm.at[didx_row], td, sem)
            return h1, h2

        def drain(ts, td, sem):
            pltpu.make_async_copy(tsrc_hbm.at[pl.ds(0, CHUNK)], ts, sem).wait()
            pltpu.make_async_copy(tdst_hbm.at[pl.ds(0, CHUNK)], td, sem).wait()

        def drain_scatter(ts, sem):
            pltpu.make_async_copy(tsrc_hbm.at[pl.ds(0, CHUNK)], ts, sem).wait()

        def compute(ts, td, didx_row, semS):
            @plsc.parallel_loop(0, CHUNK, 1, unroll=unroll)
            def edge_body(e):
                compute_edge(ts, td, e)
            return pltpu.async_copy(ts, acc.at[didx_row], semS, add=True)

        gather(idxSA.at[0], idxDA.at[0], tsA, tdA, semA)
        pltpu.async_copy(tsrc_hbm.at[pl.ds(0, CHUNK)], tsB, semSB)

        def quad_body(j, _):
            q0 = 4 * j
            fb1 = pltpu.async_copy(src2_hbm.at[pl.ds(row0 + q0 + 2, 2)],
                                   idxSB, semI)
            fb2 = pltpu.async_copy(dst2_hbm.at[pl.ds(row0 + q0 + 2, 2)],
                                   idxDB, semI)
            drain(tsA, tdA, semA)
            drain_scatter(tsB, semSB)
            g1a, g1b = gather(idxSA.at[1], idxDA.at[1], tsB, tdB, semB)
            sA0 = compute(tsA, tdA, idxDA.at[0], semSA)
            g1a.wait(); g1b.wait()
            fb1.wait(); fb2.wait()
            sA0.wait()
            g2a, g2b = gather(idxSB.at[0], idxDB.at[0], tsA, tdA, semA)
            sB1 = compute(tsB, tdB, idxDA.at[1], semSB)
            fa1 = pltpu.async_copy(src2_hbm.at[pl.ds(row0 + q0 + 4, 2)],
                                   idxSA, semI)
            fa2 = pltpu.async_copy(dst2_hbm.at[pl.ds(row0 + q0 + 4, 2)],
                                   idxDA, semI)
            g2a.wait(); g2b.wait()
            sB1.wait()
            g3a, g3b = gather(idxSB.at[1], idxDB.at[1], tsB, tdB, semB)
            sA2 = compute(tsA, tdA, idxDB.at[0], semSA)
            fa1.wait(); fa2.wait()
            g3a.wait(); g3b.wait()
            sA2.wait()
            gather(idxSA.at[0], idxDA.at[0], tsA, tdA, semA)
            compute(tsB, tdB, idxDB.at[1], semSB)
            return ()

        lax.fori_loop(0, nq, quad_body, ())
        drain(tsA, tdA, semA)
        drain_scatter(tsB, semSB)
        plsc.subcore_barrier()
        pltpu.sync_copy(acc.at[pl.ds(s * ZROWS, ZROWS)],
                        out_hbm.at[c, pl.ds(s * ZROWS, ZROWS)])

    def call(tsrc, tdst, src2, dst2, zeros):
        mesh = plsc.VectorSubcoreMesh(core_axis_name="c", subcore_axis_name="s")
        return pl.kernel(
            body,
            out_type=jax.ShapeDtypeStruct((NC, NACC, ws), jnp.float32),
            mesh=mesh,
            compiler_params=pltpu.CompilerParams(use_tc_tiling_on_sc=False),
            scratch_types=[
                pltpu.VMEM((2, CHUNK), jnp.int32),
                pltpu.VMEM((2, CHUNK), jnp.int32),
                pltpu.VMEM((2, CHUNK), jnp.int32),
                pltpu.VMEM((2, CHUNK), jnp.int32),
                pltpu.VMEM((CHUNK, ws), jnp.float32),
                pltpu.VMEM((CHUNK, wd), jnp.float32),
                pltpu.VMEM((CHUNK, ws), jnp.float32),
                pltpu.VMEM((CHUNK, wd), jnp.float32),
                pltpu.VMEM_SHARED((NACC, ws), jnp.float32),
                pltpu.SemaphoreType.DMA,
                pltpu.SemaphoreType.DMA,
                pltpu.SemaphoreType.DMA,
                pltpu.SemaphoreType.DMA,
                pltpu.SemaphoreType.DMA,
            ],
        )(tsrc, tdst, src2, dst2, zeros)

    return call


_KIDX = None


def _edge1(ts, td, e):
    av = ts[e, pl.ds(D_FEAT, 16)] + td[e, :]
    ev = jnp.where(av > 0, av, 0.2 * av)
    w = jnp.exp(ev)
    for k in range(HEADS1):
        kidx = jnp.full((16,), k, jnp.int32)
        wk = w.at[kidx].get(mode="promise_in_bounds")
        ts[e, pl.ds(k * 16, 16)] = ts[e, pl.ds(k * 16, 16)] * wk
    ts[e, pl.ds(D_FEAT, 16)] = w


def _edge2(ts, td, e):
    sv = ts[e, :]
    dv = td[e, :]
    k8 = jnp.full((16,), 8, jnp.int32)
    k9 = jnp.full((16,), 9, jnp.int32)
    a8 = sv.at[k8].get(mode="promise_in_bounds")
    a9 = dv.at[k9].get(mode="promise_in_bounds")
    av = a8 + a9
    ev = jnp.where(av > 0, av, 0.2 * av)
    w = jnp.exp(ev)
    ts[e, :] = sv * w


_sc1_call = _make_edge_pass(W1COLS, 16, _edge1, 4, nch0=128, nch1=72)
_sc2_call = _make_edge_pass(W2COLS, W2COLS, _edge2, 8, nch0=108, nch1=92)


def _sc1(t1s, t1d, src2, dst2, zeros1):
    return _sc1_call(t1s, t1d, src2, dst2, zeros1)


def _sc2(t2, src2, dst2, zeros2):
    return _sc2_call(t2, t2, src2, dst2, zeros2)


def kernel(x, edge_index, W1, a_src1, a_dst1, b1, W2, a_src2, a_dst2, b2):
    n = N_NODES
    loops = jnp.arange(n, dtype=jnp.int32)
    npad = IDX_ROWS * CHUNK - N_EDGES - n
    dummy = jnp.full((npad,), n, jnp.int32)
    src = jnp.concatenate([edge_index[0].astype(jnp.int32), loops, dummy])
    dst = jnp.concatenate([edge_index[1].astype(jnp.int32), loops, dummy])
    src = src.reshape(IDX_ROWS, CHUNK)
    dst = dst.reshape(IDX_ROWS, CHUNK)

    rows = jnp.arange(D_FEAT)
    heads = rows // OUT1
    sel = (jnp.arange(16)[None, :] == heads[:, None]).astype(jnp.float32)
    as1p = sel * a_src1.reshape(-1)[:, None]
    ad1p = sel * a_dst1.reshape(-1)[:, None]

    nodes = _tc_a(x, W1, as1p, ad1p)
    t1s = jnp.pad(nodes[:, :W1COLS], ((0, 1), (0, 0)))
    t1d = jnp.pad(nodes[:, W1COLS:160], ((0, 1), (0, 0)))

    zeros1 = jnp.zeros((ZROWS, W1COLS), jnp.float32)
    acc1 = _sc1(t1s, t1d, src, dst, zeros1)

    expand = (jnp.arange(D_FEAT)[None, :] // OUT1 ==
              jnp.arange(HEADS1)[:, None]).astype(jnp.float32)
    w2p = jnp.pad(W2, ((0, 0), (0, 16 - N_CLASSES)))
    a2s = jnp.pad(a_src2, ((0, 0), (0, 16 - N_CLASSES)))
    a2d = jnp.pad(a_dst2, ((0, 0), (0, 16 - N_CLASSES)))
    t2 = _tc_b(acc1, expand, b1.reshape(1, -1), w2p, a2s, a2d)

    zeros2 = jnp.zeros((ZROWS, W2COLS), jnp.float32)
    acc2 = _sc2(t2[:n + 1], src, dst, zeros2)

    b2p = jnp.pad(b2, (0, 16 - N_CLASSES)).reshape(1, 16)
    out = _tc_c(acc2, b2p)
    return out[:n, :N_CLASSES]


# speedup vs baseline: 1.1322x; 1.0030x over previous
import functools

import jax
import jax.numpy as jnp
from jax import lax
from jax.experimental import pallas as pl
from jax.experimental.pallas import tpu as pltpu
from jax.experimental.pallas import tpu_sc as plsc

N_NODES = 10000
N_EDGES = 320000
D_FEAT = 128
HEADS1 = 8
OUT1 = 16
N_CLASSES = 7

NC = 2
NS = 16
CHUNK = 104
NCHUNK = 100
PER_TILE = NCHUNK * CHUNK
PER_CORE = NS * PER_TILE
EDGES_PAD = NC * PER_CORE
IDX_ROWS = EDGES_PAD // CHUNK + 2
ZROWS = 632
NACC = NS * ZROWS
W1COLS = 144
W2COLS = 16


def _tc_a_body(x_ref, w1_ref, as_ref, ad_ref, out_ref):
    h = jnp.dot(x_ref[...], w1_ref[...], preferred_element_type=jnp.float32)
    asv = jnp.dot(h, as_ref[...], preferred_element_type=jnp.float32)
    adv = jnp.dot(h, ad_ref[...], preferred_element_type=jnp.float32)
    out_ref[...] = jnp.concatenate([h, asv, adv], axis=1)


def _tc_a(x, w1, as1p, ad1p):
    bn = 2000
    grid = N_NODES // bn
    return pl.pallas_call(
        _tc_a_body,
        grid=(grid,),
        in_specs=[
            pl.BlockSpec((bn, D_FEAT), lambda i: (i, 0)),
            pl.BlockSpec((D_FEAT, D_FEAT), lambda i: (0, 0)),
            pl.BlockSpec((D_FEAT, 16), lambda i: (0, 0)),
            pl.BlockSpec((D_FEAT, 16), lambda i: (0, 0)),
        ],
        out_specs=pl.BlockSpec((bn, 160), lambda i: (i, 0)),
        out_shape=jax.ShapeDtypeStruct((N_NODES, 160), jnp.float32),
    )(x, w1, as1p, ad1p)


def _tc_b_body(acc_ref, expand_ref, b1_ref, w2_ref, a2s_ref, a2d_ref, out_ref):
    s = acc_ref[0] + acc_ref[1]
    num = s[:, :D_FEAT]
    den = s[:, D_FEAT:D_FEAT + HEADS1]
    deninv = 1.0 / (den + 1e-16)
    den128 = jnp.dot(deninv, expand_ref[...],
                     preferred_element_type=jnp.float32)
    o1 = num * den128 + b1_ref[...]
    hmid = jnp.where(o1 > 0, o1, jnp.exp(o1) - 1.0)
    h2 = jnp.dot(hmid, w2_ref[...], preferred_element_type=jnp.float32)
    as2 = jnp.sum(h2 * a2s_ref[...], axis=1, keepdims=True)
    ad2 = jnp.sum(h2 * a2d_ref[...], axis=1, keepdims=True)
    col = lax.broadcasted_iota(jnp.int32, h2.shape, 1)
    t2row = (h2 + (col == 7).astype(jnp.float32)
             + as2 * (col == 8) + ad2 * (col == 9))
    row = (lax.broadcasted_iota(jnp.int32, h2.shape, 0)
           + pl.program_id(0) * h2.shape[0])
    out_ref[...] = jnp.where(row < N_NODES, t2row, 0.0)


def _tc_b(acc1, expand, b1, w2p, a2s, a2d):
    bn = 2528
    grid = NACC // bn
    return pl.pallas_call(
        _tc_b_body,
        grid=(grid,),
        in_specs=[
            pl.BlockSpec((2, bn, W1COLS), lambda i: (0, i, 0)),
            pl.BlockSpec((HEADS1, D_FEAT), lambda i: (0, 0)),
            pl.BlockSpec((1, D_FEAT), lambda i: (0, 0)),
            pl.BlockSpec((D_FEAT, 16), lambda i: (0, 0)),
            pl.BlockSpec((1, 16), lambda i: (0, 0)),
            pl.BlockSpec((1, 16), lambda i: (0, 0)),
        ],
        out_specs=pl.BlockSpec((bn, 16), lambda i: (i, 0)),
        out_shape=jax.ShapeDtypeStruct((NACC, 16), jnp.float32),
    )(acc1, expand, b1, w2p, a2s, a2d)


def _tc_c_body(acc_ref, b2_ref, out_ref):
    s = acc_ref[0] + acc_ref[1]
    den = s[:, 7:8]
    o = s * (1.0 / (den + 1e-16)) + b2_ref[...]
    col = lax.broadcasted_iota(jnp.int32, o.shape, 1)
    valid = col < N_CLASSES
    om = jnp.where(valid, o, -jnp.inf)
    m = jnp.max(om, axis=1, keepdims=True)
    ex = jnp.where(valid, jnp.exp(o - m), 0.0)
    lse = jnp.log(jnp.sum(ex, axis=1, keepdims=True))
    out_ref[...] = (o - m) - lse


def _tc_c(acc2, b2p):
    bn = 2528
    grid = NACC // bn
    return pl.pallas_call(
        _tc_c_body,
        grid=(grid,),
        in_specs=[
            pl.BlockSpec((2, bn, W2COLS), lambda i: (0, i, 0)),
            pl.BlockSpec((1, 16), lambda i: (0, 0)),
        ],
        out_specs=pl.BlockSpec((bn, 16), lambda i: (i, 0)),
        out_shape=jax.ShapeDtypeStruct((NACC, 16), jnp.float32),
    )(acc2, b2p)


def _make_edge_pass(ws, wd, compute_edge, unroll, nch0=NCHUNK, nch1=NCHUNK):
    def body(tsrc_hbm, tdst_hbm, src2_hbm, dst2_hbm, zeros_hbm, out_hbm,
             idxSA, idxDA, idxSB, idxDB, tsA, tdA, tsB, tdB,
             acc, semA, semB, semI, semSA, semSB):
        c = lax.axis_index("c")
        s = lax.axis_index("s")
        row0 = jnp.where(c == 0, s * nch0, NS * nch0 + s * nch1)
        nq = jnp.where(c == 0, nch0 // 4, nch1 // 4)

        pltpu.sync_copy(zeros_hbm, acc.at[pl.ds(s * ZROWS, ZROWS)])
        pltpu.sync_copy(src2_hbm.at[pl.ds(row0, 2)], idxSA)
        pltpu.sync_copy(dst2_hbm.at[pl.ds(row0, 2)], idxDA)
        plsc.subcore_barrier()

        def gather(sidx_row, didx_row, ts, td, sem):
            h1 = pltpu.async_copy(tsrc_hbm.at[sidx_row], ts, sem)
            h2 = pltpu.async_copy(tdst_hbm.at[didx_row], td, sem)
            return h1, h2

        def drain(ts, td, sem):
            pltpu.make_async_copy(tsrc_hbm.at[pl.ds(0, CHUNK)], ts, sem).wait()
            pltpu.make_async_copy(tdst_hbm.at[pl.ds(0, CHUNK)], td, sem).wait()

        def drain_scatter(ts, sem):
            pltpu.make_async_copy(tsrc_hbm.at[pl.ds(0, CHUNK)], ts, sem).wait()

        def compute(ts, td, didx_row, semS):
            @plsc.parallel_loop(0, CHUNK, 1, unroll=unroll)
            def edge_body(e):
                compute_edge(ts, td, e)
            return pltpu.async_copy(ts, acc.at[didx_row], semS, add=True)

        gather(idxSA.at[0], idxDA.at[0], tsA, tdA, semA)
        pltpu.async_copy(tsrc_hbm.at[pl.ds(0, CHUNK)], tsB, semSB)

        def quad_body(j, _):
            q0 = 4 * j
            fb1 = pltpu.async_copy(src2_hbm.at[pl.ds(row0 + q0 + 2, 2)],
                                   idxSB, semI)
            fb2 = pltpu.async_copy(dst2_hbm.at[pl.ds(row0 + q0 + 2, 2)],
                                   idxDB, semI)
            drain(tsA, tdA, semA)
            drain_scatter(tsB, semSB)
            g1a, g1b = gather(idxSA.at[1], idxDA.at[1], tsB, tdB, semB)
            sA0 = compute(tsA, tdA, idxDA.at[0], semSA)
            g1a.wait(); g1b.wait()
            fb1.wait(); fb2.wait()
            sA0.wait()
            g2a, g2b = gather(idxSB.at[0], idxDB.at[0], tsA, tdA, semA)
            sB1 = compute(tsB, tdB, idxDA.at[1], semSB)
            fa1 = pltpu.async_copy(src2_hbm.at[pl.ds(row0 + q0 + 4, 2)],
                                   idxSA, semI)
            fa2 = pltpu.async_copy(dst2_hbm.at[pl.ds(row0 + q0 + 4, 2)],
                                   idxDA, semI)
            g2a.wait(); g2b.wait()
            sB1.wait()
            g3a, g3b = gather(idxSB.at[1], idxDB.at[1], tsB, tdB, semB)
            sA2 = compute(tsA, tdA, idxDB.at[0], semSA)
            fa1.wait(); fa2.wait()
            g3a.wait(); g3b.wait()
            sA2.wait()
            gather(idxSA.at[0], idxDA.at[0], tsA, tdA, semA)
            compute(tsB, tdB, idxDB.at[1], semSB)
            return ()

        lax.fori_loop(0, nq, quad_body, ())
        drain(tsA, tdA, semA)
        drain_scatter(tsB, semSB)
        plsc.subcore_barrier()
        pltpu.sync_copy(acc.at[pl.ds(s * ZROWS, ZROWS)],
                        out_hbm.at[c, pl.ds(s * ZROWS, ZROWS)])

    def call(tsrc, tdst, src2, dst2, zeros):
        mesh = plsc.VectorSubcoreMesh(core_axis_name="c", subcore_axis_name="s")
        return pl.kernel(
            body,
            out_type=jax.ShapeDtypeStruct((NC, NACC, ws), jnp.float32),
            mesh=mesh,
            compiler_params=pltpu.CompilerParams(use_tc_tiling_on_sc=False),
            scratch_types=[
                pltpu.VMEM((2, CHUNK), jnp.int32),
                pltpu.VMEM((2, CHUNK), jnp.int32),
                pltpu.VMEM((2, CHUNK), jnp.int32),
                pltpu.VMEM((2, CHUNK), jnp.int32),
                pltpu.VMEM((CHUNK, ws), jnp.float32),
                pltpu.VMEM((CHUNK, wd), jnp.float32),
                pltpu.VMEM((CHUNK, ws), jnp.float32),
                pltpu.VMEM((CHUNK, wd), jnp.float32),
                pltpu.VMEM_SHARED((NACC, ws), jnp.float32),
                pltpu.SemaphoreType.DMA,
                pltpu.SemaphoreType.DMA,
                pltpu.SemaphoreType.DMA,
                pltpu.SemaphoreType.DMA,
                pltpu.SemaphoreType.DMA,
            ],
        )(tsrc, tdst, src2, dst2, zeros)

    return call


_KIDX = None


def _edge1(ts, td, e):
    av = ts[e, pl.ds(D_FEAT, 16)] + td[e, :]
    ev = jnp.where(av > 0, av, 0.2 * av)
    w = jnp.exp(ev)
    for k in range(HEADS1):
        kidx = jnp.full((16,), k, jnp.int32)
        wk = w.at[kidx].get(mode="promise_in_bounds")
        ts[e, pl.ds(k * 16, 16)] = ts[e, pl.ds(k * 16, 16)] * wk
    ts[e, pl.ds(D_FEAT, 16)] = w


def _edge2(ts, td, e):
    sv = ts[e, :]
    dv = td[e, :]
    k8 = jnp.full((16,), 8, jnp.int32)
    k9 = jnp.full((16,), 9, jnp.int32)
    a8 = sv.at[k8].get(mode="promise_in_bounds")
    a9 = dv.at[k9].get(mode="promise_in_bounds")
    av = a8 + a9
    ev = jnp.where(av > 0, av, 0.2 * av)
    w = jnp.exp(ev)
    ts[e, :] = sv * w


_sc1_call = _make_edge_pass(W1COLS, 16, _edge1, 4, nch0=136, nch1=64)
_sc2_call = _make_edge_pass(W2COLS, W2COLS, _edge2, 8, nch0=108, nch1=92)


def _sc1(t1s, t1d, src2, dst2, zeros1):
    return _sc1_call(t1s, t1d, src2, dst2, zeros1)


def _sc2(t2, src2, dst2, zeros2):
    return _sc2_call(t2, t2, src2, dst2, zeros2)


def kernel(x, edge_index, W1, a_src1, a_dst1, b1, W2, a_src2, a_dst2, b2):
    n = N_NODES
    loops = jnp.arange(n, dtype=jnp.int32)
    npad = IDX_ROWS * CHUNK - N_EDGES - n
    dummy = jnp.full((npad,), n, jnp.int32)
    src = jnp.concatenate([edge_index[0].astype(jnp.int32), loops, dummy])
    dst = jnp.concatenate([edge_index[1].astype(jnp.int32), loops, dummy])
    src = src.reshape(IDX_ROWS, CHUNK)
    dst = dst.reshape(IDX_ROWS, CHUNK)

    rows = jnp.arange(D_FEAT)
    heads = rows // OUT1
    sel = (jnp.arange(16)[None, :] == heads[:, None]).astype(jnp.float32)
    as1p = sel * a_src1.reshape(-1)[:, None]
    ad1p = sel * a_dst1.reshape(-1)[:, None]

    nodes = _tc_a(x, W1, as1p, ad1p)
    t1s = jnp.pad(nodes[:, :W1COLS], ((0, 1), (0, 0)))
    t1d = jnp.pad(nodes[:, W1COLS:160], ((0, 1), (0, 0)))

    zeros1 = jnp.zeros((ZROWS, W1COLS), jnp.float32)
    acc1 = _sc1(t1s, t1d, src, dst, zeros1)

    expand = (jnp.arange(D_FEAT)[None, :] // OUT1 ==
              jnp.arange(HEADS1)[:, None]).astype(jnp.float32)
    w2p = jnp.pad(W2, ((0, 0), (0, 16 - N_CLASSES)))
    a2s = jnp.pad(a_src2, ((0, 0), (0, 16 - N_CLASSES)))
    a2d = jnp.pad(a_dst2, ((0, 0), (0, 16 - N_CLASSES)))
    t2 = _tc_b(acc1, expand, b1.reshape(1, -1), w2p, a2s, a2d)

    zeros2 = jnp.zeros((ZROWS, W2COLS), jnp.float32)
    acc2 = _sc2(t2[:n + 1], src, dst, zeros2)

    b2p = jnp.pad(b2, (0, 16 - N_CLASSES)).reshape(1, 16)
    out = _tc_c(acc2, b2p)
    return out[:n, :N_CLASSES]


# speedup vs baseline: 1.1473x; 1.0134x over previous
import functools

import jax
import jax.numpy as jnp
from jax import lax
from jax.experimental import pallas as pl
from jax.experimental.pallas import tpu as pltpu
from jax.experimental.pallas import tpu_sc as plsc

N_NODES = 10000
N_EDGES = 320000
D_FEAT = 128
HEADS1 = 8
OUT1 = 16
N_CLASSES = 7

NC = 2
NS = 16
CHUNK = 104
NCHUNK = 100
PER_TILE = NCHUNK * CHUNK
PER_CORE = NS * PER_TILE
EDGES_PAD = NC * PER_CORE
IDX_ROWS = EDGES_PAD // CHUNK + 2
ZROWS = 632
NACC = NS * ZROWS
W1COLS = 144
W2COLS = 16


def _tc_a_body(x_ref, w1_ref, as_ref, ad_ref, ts_ref, td_ref):
    h = jnp.dot(x_ref[...], w1_ref[...], preferred_element_type=jnp.float32)
    asv = jnp.dot(h, as_ref[...], preferred_element_type=jnp.float32)
    adv = jnp.dot(h, ad_ref[...], preferred_element_type=jnp.float32)
    ts_ref[...] = jnp.concatenate([h, asv], axis=1)
    td_ref[...] = adv


def _tc_a(x, w1, as1p, ad1p):
    bn = 2000
    grid = N_NODES // bn
    return pl.pallas_call(
        _tc_a_body,
        grid=(grid,),
        in_specs=[
            pl.BlockSpec((bn, D_FEAT), lambda i: (i, 0)),
            pl.BlockSpec((D_FEAT, D_FEAT), lambda i: (0, 0)),
            pl.BlockSpec((D_FEAT, 16), lambda i: (0, 0)),
            pl.BlockSpec((D_FEAT, 16), lambda i: (0, 0)),
        ],
        out_specs=[
            pl.BlockSpec((bn, W1COLS), lambda i: (i, 0)),
            pl.BlockSpec((bn, 16), lambda i: (i, 0)),
        ],
        out_shape=[
            jax.ShapeDtypeStruct((N_NODES + 8, W1COLS), jnp.float32),
            jax.ShapeDtypeStruct((N_NODES + 8, 16), jnp.float32),
        ],
    )(x, w1, as1p, ad1p)


def _tc_b_body(acc_ref, expand_ref, b1_ref, w2_ref, a2s_ref, a2d_ref, out_ref):
    s = acc_ref[0] + acc_ref[1]
    num = s[:, :D_FEAT]
    den = s[:, D_FEAT:D_FEAT + HEADS1]
    deninv = 1.0 / (den + 1e-16)
    den128 = jnp.dot(deninv, expand_ref[...],
                     preferred_element_type=jnp.float32)
    o1 = num * den128 + b1_ref[...]
    hmid = jnp.where(o1 > 0, o1, jnp.exp(o1) - 1.0)
    h2 = jnp.dot(hmid, w2_ref[...], preferred_element_type=jnp.float32)
    as2 = jnp.sum(h2 * a2s_ref[...], axis=1, keepdims=True)
    ad2 = jnp.sum(h2 * a2d_ref[...], axis=1, keepdims=True)
    col = lax.broadcasted_iota(jnp.int32, h2.shape, 1)
    t2row = (h2 + (col == 7).astype(jnp.float32)
             + as2 * (col == 8) + ad2 * (col == 9))
    row = (lax.broadcasted_iota(jnp.int32, h2.shape, 0)
           + pl.program_id(0) * h2.shape[0])
    out_ref[...] = jnp.where(row < N_NODES, t2row, 0.0)


def _tc_b(acc1, expand, b1, w2p, a2s, a2d):
    bn = 2528
    grid = NACC // bn
    return pl.pallas_call(
        _tc_b_body,
        grid=(grid,),
        in_specs=[
            pl.BlockSpec((2, bn, W1COLS), lambda i: (0, i, 0)),
            pl.BlockSpec((HEADS1, D_FEAT), lambda i: (0, 0)),
            pl.BlockSpec((1, D_FEAT), lambda i: (0, 0)),
            pl.BlockSpec((D_FEAT, 16), lambda i: (0, 0)),
            pl.BlockSpec((1, 16), lambda i: (0, 0)),
            pl.BlockSpec((1, 16), lambda i: (0, 0)),
        ],
        out_specs=pl.BlockSpec((bn, 16), lambda i: (i, 0)),
        out_shape=jax.ShapeDtypeStruct((NACC, 16), jnp.float32),
    )(acc1, expand, b1, w2p, a2s, a2d)


def _tc_c_body(acc_ref, b2_ref, out_ref):
    s = acc_ref[0] + acc_ref[1]
    den = s[:, 7:8]
    o = s * (1.0 / (den + 1e-16)) + b2_ref[...]
    col = lax.broadcasted_iota(jnp.int32, o.shape, 1)
    valid = col < N_CLASSES
    om = jnp.where(valid, o, -jnp.inf)
    m = jnp.max(om, axis=1, keepdims=True)
    ex = jnp.where(valid, jnp.exp(o - m), 0.0)
    lse = jnp.log(jnp.sum(ex, axis=1, keepdims=True))
    out_ref[...] = (o - m) - lse


def _tc_c(acc2, b2p):
    bn = 2528
    grid = NACC // bn
    return pl.pallas_call(
        _tc_c_body,
        grid=(grid,),
        in_specs=[
            pl.BlockSpec((2, bn, W2COLS), lambda i: (0, i, 0)),
            pl.BlockSpec((1, 16), lambda i: (0, 0)),
        ],
        out_specs=pl.BlockSpec((bn, 16), lambda i: (i, 0)),
        out_shape=jax.ShapeDtypeStruct((NACC, 16), jnp.float32),
    )(acc2, b2p)


def _make_edge_pass(ws, wd, compute_edge, unroll, nch0=NCHUNK, nch1=NCHUNK):
    def body(tsrc_hbm, tdst_hbm, src2_hbm, dst2_hbm, zeros_hbm, out_hbm,
             idxSA, idxDA, idxSB, idxDB, tsA, tdA, tsB, tdB,
             acc, semA, semB, semI, semSA, semSB):
        c = lax.axis_index("c")
        s = lax.axis_index("s")
        row0 = jnp.where(c == 0, s * nch0, NS * nch0 + s * nch1)
        nq = jnp.where(c == 0, nch0 // 4, nch1 // 4)

        pltpu.sync_copy(zeros_hbm, acc.at[pl.ds(s * ZROWS, ZROWS)])
        pltpu.sync_copy(src2_hbm.at[pl.ds(row0, 2)], idxSA)
        pltpu.sync_copy(dst2_hbm.at[pl.ds(row0, 2)], idxDA)
        plsc.subcore_barrier()

        def gather(sidx_row, didx_row, ts, td, sem):
            h1 = pltpu.async_copy(tsrc_hbm.at[sidx_row], ts, sem)
            h2 = pltpu.async_copy(tdst_hbm.at[didx_row], td, sem)
            return h1, h2

        def drain(ts, td, sem):
            pltpu.make_async_copy(tsrc_hbm.at[pl.ds(0, CHUNK)], ts, sem).wait()
            pltpu.make_async_copy(tdst_hbm.at[pl.ds(0, CHUNK)], td, sem).wait()

        def drain_scatter(ts, sem):
            pltpu.make_async_copy(tsrc_hbm.at[pl.ds(0, CHUNK)], ts, sem).wait()

        def compute(ts, td, didx_row, semS):
            @plsc.parallel_loop(0, CHUNK, 1, unroll=unroll)
            def edge_body(e):
                compute_edge(ts, td, e)
            return pltpu.async_copy(ts, acc.at[didx_row], semS, add=True)

        gather(idxSA.at[0], idxDA.at[0], tsA, tdA, semA)
        pltpu.async_copy(tsrc_hbm.at[pl.ds(0, CHUNK)], tsB, semSB)

        def quad_body(j, _):
            q0 = 4 * j
            fb1 = pltpu.async_copy(src2_hbm.at[pl.ds(row0 + q0 + 2, 2)],
                                   idxSB, semI)
            fb2 = pltpu.async_copy(dst2_hbm.at[pl.ds(row0 + q0 + 2, 2)],
                                   idxDB, semI)
            drain(tsA, tdA, semA)
            drain_scatter(tsB, semSB)
            g1a, g1b = gather(idxSA.at[1], idxDA.at[1], tsB, tdB, semB)
            sA0 = compute(tsA, tdA, idxDA.at[0], semSA)
            g1a.wait(); g1b.wait()
            fb1.wait(); fb2.wait()
            sA0.wait()
            g2a, g2b = gather(idxSB.at[0], idxDB.at[0], tsA, tdA, semA)
            sB1 = compute(tsB, tdB, idxDA.at[1], semSB)
            fa1 = pltpu.async_copy(src2_hbm.at[pl.ds(row0 + q0 + 4, 2)],
                                   idxSA, semI)
            fa2 = pltpu.async_copy(dst2_hbm.at[pl.ds(row0 + q0 + 4, 2)],
                                   idxDA, semI)
            g2a.wait(); g2b.wait()
            sB1.wait()
            g3a, g3b = gather(idxSB.at[1], idxDB.at[1], tsB, tdB, semB)
            sA2 = compute(tsA, tdA, idxDB.at[0], semSA)
            fa1.wait(); fa2.wait()
            g3a.wait(); g3b.wait()
            sA2.wait()
            gather(idxSA.at[0], idxDA.at[0], tsA, tdA, semA)
            compute(tsB, tdB, idxDB.at[1], semSB)
            return ()

        lax.fori_loop(0, nq, quad_body, ())
        drain(tsA, tdA, semA)
        drain_scatter(tsB, semSB)
        plsc.subcore_barrier()
        pltpu.sync_copy(acc.at[pl.ds(s * ZROWS, ZROWS)],
                        out_hbm.at[c, pl.ds(s * ZROWS, ZROWS)])

    def call(tsrc, tdst, src2, dst2, zeros):
        mesh = plsc.VectorSubcoreMesh(core_axis_name="c", subcore_axis_name="s")
        return pl.kernel(
            body,
            out_type=jax.ShapeDtypeStruct((NC, NACC, ws), jnp.float32),
            mesh=mesh,
            compiler_params=pltpu.CompilerParams(use_tc_tiling_on_sc=False),
            scratch_types=[
                pltpu.VMEM((2, CHUNK), jnp.int32),
                pltpu.VMEM((2, CHUNK), jnp.int32),
                pltpu.VMEM((2, CHUNK), jnp.int32),
                pltpu.VMEM((2, CHUNK), jnp.int32),
                pltpu.VMEM((CHUNK, ws), jnp.float32),
                pltpu.VMEM((CHUNK, wd), jnp.float32),
                pltpu.VMEM((CHUNK, ws), jnp.float32),
                pltpu.VMEM((CHUNK, wd), jnp.float32),
                pltpu.VMEM_SHARED((NACC, ws), jnp.float32),
                pltpu.SemaphoreType.DMA,
                pltpu.SemaphoreType.DMA,
                pltpu.SemaphoreType.DMA,
                pltpu.SemaphoreType.DMA,
                pltpu.SemaphoreType.DMA,
            ],
        )(tsrc, tdst, src2, dst2, zeros)

    return call


_KIDX = None


def _edge1(ts, td, e):
    av = ts[e, pl.ds(D_FEAT, 16)] + td[e, :]
    ev = jnp.where(av > 0, av, 0.2 * av)
    w = jnp.exp(ev)
    for k in range(HEADS1):
        kidx = jnp.full((16,), k, jnp.int32)
        wk = w.at[kidx].get(mode="promise_in_bounds")
        ts[e, pl.ds(k * 16, 16)] = ts[e, pl.ds(k * 16, 16)] * wk
    ts[e, pl.ds(D_FEAT, 16)] = w


def _edge2(ts, td, e):
    sv = ts[e, :]
    dv = td[e, :]
    k8 = jnp.full((16,), 8, jnp.int32)
    k9 = jnp.full((16,), 9, jnp.int32)
    a8 = sv.at[k8].get(mode="promise_in_bounds")
    a9 = dv.at[k9].get(mode="promise_in_bounds")
    av = a8 + a9
    ev = jnp.where(av > 0, av, 0.2 * av)
    w = jnp.exp(ev)
    ts[e, :] = sv * w


_sc1_call = _make_edge_pass(W1COLS, 16, _edge1, 4, nch0=136, nch1=64)
_sc2_call = _make_edge_pass(W2COLS, W2COLS, _edge2, 8, nch0=108, nch1=92)


def _sc1(t1s, t1d, src2, dst2, zeros1):
    return _sc1_call(t1s, t1d, src2, dst2, zeros1)


def _sc2(t2, src2, dst2, zeros2):
    return _sc2_call(t2, t2, src2, dst2, zeros2)


def kernel(x, edge_index, W1, a_src1, a_dst1, b1, W2, a_src2, a_dst2, b2):
    n = N_NODES
    loops = jnp.arange(n, dtype=jnp.int32)
    npad = IDX_ROWS * CHUNK - N_EDGES - n
    dummy = jnp.full((npad,), n, jnp.int32)
    src = jnp.concatenate([edge_index[0].astype(jnp.int32), loops, dummy])
    dst = jnp.concatenate([edge_index[1].astype(jnp.int32), loops, dummy])
    src = src.reshape(IDX_ROWS, CHUNK)
    dst = dst.reshape(IDX_ROWS, CHUNK)

    rows = jnp.arange(D_FEAT)
    heads = rows // OUT1
    sel = (jnp.arange(16)[None, :] == heads[:, None]).astype(jnp.float32)
    as1p = sel * a_src1.reshape(-1)[:, None]
    ad1p = sel * a_dst1.reshape(-1)[:, None]

    t1s, t1d = _tc_a(x, W1, as1p, ad1p)

    zeros1 = jnp.zeros((ZROWS, W1COLS), jnp.float32)
    acc1 = _sc1(t1s, t1d, src, dst, zeros1)

    expand = (jnp.arange(D_FEAT)[None, :] // OUT1 ==
              jnp.arange(HEADS1)[:, None]).astype(jnp.float32)
    w2p = jnp.pad(W2, ((0, 0), (0, 16 - N_CLASSES)))
    a2s = jnp.pad(a_src2, ((0, 0), (0, 16 - N_CLASSES)))
    a2d = jnp.pad(a_dst2, ((0, 0), (0, 16 - N_CLASSES)))
    t2 = _tc_b(acc1, expand, b1.reshape(1, -1), w2p, a2s, a2d)

    zeros2 = jnp.zeros((ZROWS, W2COLS), jnp.float32)
    acc2 = _sc2(t2, src, dst, zeros2)

    b2p = jnp.pad(b2, (0, 16 - N_CLASSES)).reshape(1, 16)
    out = _tc_c(acc2, b2p)
    return out[:n, :N_CLASSES]
